# Initial kernel scaffold; baseline (speedup 1.0000x reference)
import jax
import jax.numpy as jnp
from jax.experimental import pallas as pl


def kernel(x, edge_index, batch, W1, b1, W2, b2, Wfc, bfc):
    raise NotImplementedError("write your pallas kernel here")


# speedup vs baseline: 19.8846x; 19.8846x over previous
import functools
import jax
import jax.numpy as jnp
from jax import lax
from jax.experimental import pallas as pl
from jax.experimental.pallas import tpu as pltpu
from jax.experimental.pallas import tpu_sc as plsc

N = 10000
E = 320000
D = 128
G = 64
NC = 2
NS = 16
NW = NC * NS
EPW = E // NW
CH = 80
NCH = EPW // CH
NP = 10240
SEG = NP // NS
RB = 10
BR = N // RB

_mesh = plsc.VectorSubcoreMesh(core_axis_name="c", subcore_axis_name="s")


@functools.partial(
    pl.kernel,
    out_type=jax.ShapeDtypeStruct((NC * NP,), jnp.float32),
    mesh=_mesh,
    scratch_types=[
        pltpu.VMEM((NCH, CH), jnp.int32),
        pltpu.VMEM((CH,), jnp.float32),
        pltpu.VMEM((SEG,), jnp.float32),
        pltpu.VMEM_SHARED((NP,), jnp.float32),
        pltpu.SemaphoreType.DMA,
    ],
)
def _deg(dst_hbm, out_hbm, idx_v, ones_v, zero_v, acc_sh, sem):
    cid = lax.axis_index("c")
    sid = lax.axis_index("s")
    wid = sid * NC + cid
    ones16 = jnp.ones((16,), jnp.float32)
    zeros16 = jnp.zeros((16,), jnp.float32)
    for i in range(CH // 16):
        ones_v[pl.ds(i * 16, 16)] = ones16

    def zbody(i, c):
        zero_v[pl.ds(i * 16, 16)] = zeros16
        return c

    lax.fori_loop(0, SEG // 16, zbody, 0)
    pltpu.sync_copy(zero_v, acc_sh.at[pl.ds(sid * SEG, SEG)])
    pltpu.sync_copy(dst_hbm.at[wid], idx_v)
    plsc.subcore_barrier()

    def body(j, c):
        pltpu.sync_copy(ones_v, acc_sh.at[idx_v.at[j]], add=True)
        return c

    lax.fori_loop(0, NCH, body, 0)
    plsc.subcore_barrier()
    pltpu.sync_copy(acc_sh.at[pl.ds(sid * SEG, SEG)],
                    out_hbm.at[pl.ds(cid * NP + sid * SEG, SEG)])


@functools.partial(
    pl.kernel,
    out_type=jax.ShapeDtypeStruct((NC, NP, D), jnp.float32),
    mesh=_mesh,
    scratch_types=[
        pltpu.VMEM((NCH, CH), jnp.int32),
        pltpu.VMEM((NCH, CH), jnp.int32),
        pltpu.VMEM((CH, D), jnp.float32),
        pltpu.VMEM((CH, D), jnp.float32),
        pltpu.VMEM_SHARED((NP, D), jnp.float32),
        pltpu.SemaphoreType.DMA,
    ],
)
def _agg(z_hbm, src_hbm, dst_hbm, out_hbm, srcv, dstv, rows0, rows1, acc_sh, sem):
    cid = lax.axis_index("c")
    sid = lax.axis_index("s")
    wid = sid * NC + cid
    pltpu.sync_copy(src_hbm.at[wid], srcv)
    pltpu.sync_copy(dst_hbm.at[wid], dstv)

    zeros16 = jnp.zeros((16,), jnp.float32)

    def zbody(i, c):
        for k in range(D // 16):
            rows0[i, pl.ds(k * 16, 16)] = zeros16
        return c

    lax.fori_loop(0, CH, zbody, 0)
    base = sid * SEG
    for r in range(SEG // CH):
        pltpu.sync_copy(rows0, acc_sh.at[pl.ds(base + r * CH, CH)])
    plsc.subcore_barrier()

    def body(j, c):
        pltpu.async_copy(z_hbm.at[srcv.at[j]], rows0, sem).wait()
        pltpu.sync_copy(rows0, acc_sh.at[dstv.at[j]], add=True)
        return c

    lax.fori_loop(0, NCH, body, 0)
    plsc.subcore_barrier()
    pltpu.sync_copy(acc_sh.at[pl.ds(base, SEG)],
                    out_hbm.at[cid, pl.ds(base, SEG)])


def _degfin_body(dp_ref, dinv_ref):
    d = dp_ref[0:NP // D, :] + dp_ref[NP // D:2 * (NP // D), :] + 1.0
    dinv_ref[...] = lax.rsqrt(d)


def _degfin(degp2):
    return pl.pallas_call(
        _degfin_body,
        out_shape=jax.ShapeDtypeStruct((NP // D, D), jnp.float32),
    )(degp2)


def _zmat_body(x_ref, w_ref, dinv_ref, z_ref):
    xw = jnp.dot(x_ref[...], w_ref[...], preferred_element_type=jnp.float32)
    z_ref[...] = xw * dinv_ref[...]


def _zmat(x, W, dinv_col):
    return pl.pallas_call(
        _zmat_body,
        grid=(RB,),
        in_specs=[
            pl.BlockSpec((BR, D), lambda i: (i, 0)),
            pl.BlockSpec((D, D), lambda i: (0, 0)),
            pl.BlockSpec((BR, 1), lambda i: (i, 0)),
        ],
        out_specs=pl.BlockSpec((BR, D), lambda i: (i, 0)),
        out_shape=jax.ShapeDtypeStruct((N, D), jnp.float32),
    )(x, W, dinv_col)


def _comb_body(a0_ref, a1_ref, z_ref, dinv_ref, b_ref, w_ref, z2_ref):
    h = jnp.maximum(
        (a0_ref[...] + a1_ref[...] + z_ref[...]) * dinv_ref[...] + b_ref[...],
        0.0)
    z2_ref[...] = jnp.dot(h, w_ref[...],
                          preferred_element_type=jnp.float32) * dinv_ref[...]


def _comb(a0, a1, z, dinv_col, b_row, W):
    return pl.pallas_call(
        _comb_body,
        grid=(RB,),
        in_specs=[
            pl.BlockSpec((BR, D), lambda i: (i, 0)),
            pl.BlockSpec((BR, D), lambda i: (i, 0)),
            pl.BlockSpec((BR, D), lambda i: (i, 0)),
            pl.BlockSpec((BR, 1), lambda i: (i, 0)),
            pl.BlockSpec((1, D), lambda i: (0, 0)),
            pl.BlockSpec((D, D), lambda i: (0, 0)),
        ],
        out_specs=pl.BlockSpec((BR, D), lambda i: (i, 0)),
        out_shape=jax.ShapeDtypeStruct((N, D), jnp.float32),
    )(a0, a1, z, dinv_col, b_row, W)


def _fin_body(a0_ref, a1_ref, z_ref, dinv_ref, b_ref, batch_ref, wfc_ref,
              bfc_ref, logits_ref, prob_ref, pooled_s, counts_s):
    i = pl.program_id(0)

    @pl.when(i == 0)
    def _():
        pooled_s[...] = jnp.zeros_like(pooled_s)
        counts_s[...] = jnp.zeros_like(counts_s)

    h = jnp.maximum(
        (a0_ref[...] + a1_ref[...] + z_ref[...]) * dinv_ref[...] + b_ref[...],
        0.0)
    onehot = (batch_ref[...] == lax.broadcasted_iota(jnp.int32, (1, G), 1)
              ).astype(jnp.float32)
    pooled_s[...] += lax.dot_general(onehot, h, (((0,), (0,)), ((), ())),
                                     preferred_element_type=jnp.float32)
    counts_s[...] += lax.dot_general(onehot, jnp.ones((BR, 1), jnp.float32),
                                     (((0,), (0,)), ((), ())),
                                     preferred_element_type=jnp.float32)

    @pl.when(i == pl.num_programs(0) - 1)
    def _():
        pooled = pooled_s[...] / jnp.maximum(counts_s[...], 1.0)
        logits = jnp.dot(pooled, wfc_ref[...],
                         preferred_element_type=jnp.float32) + bfc_ref[...]
        logits_ref[...] = logits
        m = jnp.max(logits, axis=1, keepdims=True)
        e = jnp.exp(logits - m)
        prob_ref[...] = e / jnp.sum(e, axis=1, keepdims=True)


def _final(a0, a1, z, dinv_col, b_row, batch_col, Wfc, bfc_row):
    return pl.pallas_call(
        _fin_body,
        grid=(RB,),
        in_specs=[
            pl.BlockSpec((BR, D), lambda i: (i, 0)),
            pl.BlockSpec((BR, D), lambda i: (i, 0)),
            pl.BlockSpec((BR, D), lambda i: (i, 0)),
            pl.BlockSpec((BR, 1), lambda i: (i, 0)),
            pl.BlockSpec((1, D), lambda i: (0, 0)),
            pl.BlockSpec((BR, 1), lambda i: (i, 0)),
            pl.BlockSpec((D, 2), lambda i: (0, 0)),
            pl.BlockSpec((1, 2), lambda i: (0, 0)),
        ],
        out_specs=[
            pl.BlockSpec((G, 2), lambda i: (0, 0)),
            pl.BlockSpec((G, 2), lambda i: (0, 0)),
        ],
        out_shape=[
            jax.ShapeDtypeStruct((G, 2), jnp.float32),
            jax.ShapeDtypeStruct((G, 2), jnp.float32),
        ],
        scratch_shapes=[
            pltpu.VMEM((G, D), jnp.float32),
            pltpu.VMEM((G, 1), jnp.float32),
        ],
    )(a0, a1, z, dinv_col, b_row, batch_col, Wfc, bfc_row)


def kernel(x, edge_index, batch, W1, b1, W2, b2, Wfc, bfc):
    src3 = edge_index[0].reshape(NW, NCH, CH)
    dst3 = edge_index[1].reshape(NW, NCH, CH)

    degp = _deg(dst3)
    degp2 = degp.reshape(NC * NP // D, D)
    dinv = _degfin(degp2)
    dinv_col = dinv.reshape(NP)[:N].reshape(N, 1)

    z1 = _zmat(x, W1, dinv_col)
    accp1 = _agg(z1, src3, dst3)[:, :N, :]
    z2 = _comb(accp1[0], accp1[1], z1, dinv_col, b1.reshape(1, D), W2)
    accp2 = _agg(z2, src3, dst3)[:, :N, :]
    logits, y_prob = _final(accp2[0], accp2[1], z2, dinv_col,
                            b2.reshape(1, D), batch.reshape(N, 1),
                            Wfc, bfc.reshape(1, 2))
    return (logits, y_prob)


# speedup vs baseline: 31.2062x; 1.5694x over previous
import functools
import jax
import jax.numpy as jnp
from jax import lax
from jax.experimental import pallas as pl
from jax.experimental.pallas import tpu as pltpu
from jax.experimental.pallas import tpu_sc as plsc

N = 10000
E = 320000
D = 128
G = 64
NC = 2
NS = 16
NW = NC * NS
EPW = E // NW
CH = 80
NCH = EPW // CH
NP = 10240
SEG = NP // NS
RB = 10
BR = N // RB

_mesh = plsc.VectorSubcoreMesh(core_axis_name="c", subcore_axis_name="s")


@functools.partial(
    pl.kernel,
    out_type=jax.ShapeDtypeStruct((NC * NP,), jnp.float32),
    mesh=_mesh,
    scratch_types=[
        pltpu.VMEM((NCH, CH), jnp.int32),
        pltpu.VMEM((CH,), jnp.float32),
        pltpu.VMEM((SEG,), jnp.float32),
        pltpu.VMEM_SHARED((NP,), jnp.float32),
        pltpu.SemaphoreType.DMA,
    ],
)
def _deg(dst_hbm, out_hbm, idx_v, ones_v, zero_v, acc_sh, sem):
    cid = lax.axis_index("c")
    sid = lax.axis_index("s")
    wid = sid * NC + cid
    ones16 = jnp.ones((16,), jnp.float32)
    zeros16 = jnp.zeros((16,), jnp.float32)
    for i in range(CH // 16):
        ones_v[pl.ds(i * 16, 16)] = ones16

    def zbody(i, c):
        zero_v[pl.ds(i * 16, 16)] = zeros16
        return c

    lax.fori_loop(0, SEG // 16, zbody, 0)
    pltpu.sync_copy(zero_v, acc_sh.at[pl.ds(sid * SEG, SEG)])
    pltpu.sync_copy(dst_hbm.at[wid], idx_v)
    plsc.subcore_barrier()

    def body(j, c):
        pltpu.sync_copy(ones_v, acc_sh.at[idx_v.at[j]], add=True)
        return c

    lax.fori_loop(0, NCH, body, 0)
    plsc.subcore_barrier()
    pltpu.sync_copy(acc_sh.at[pl.ds(sid * SEG, SEG)],
                    out_hbm.at[pl.ds(cid * NP + sid * SEG, SEG)])


@functools.partial(
    pl.kernel,
    out_type=jax.ShapeDtypeStruct((NC, NP, D), jnp.float32),
    mesh=_mesh,
    scratch_types=[
        pltpu.VMEM((EPW,), jnp.int32),
        pltpu.VMEM((NCH, CH), jnp.int32),
        pltpu.VMEM((CH, D), jnp.float32),
        pltpu.VMEM((CH, D), jnp.float32),
        pltpu.VMEM_SHARED((NP, D), jnp.float32),
        pltpu.SemaphoreType.DMA,
    ],
)
def _agg(z_hbm, src_hbm, dst_hbm, out_hbm, srcv, dstv, rows0, rows1, acc_sh,
         sem0):
    cid = lax.axis_index("c")
    sid = lax.axis_index("s")
    wid = sid * NC + cid
    pltpu.async_copy(src_hbm.at[wid], srcv, sem0)
    pltpu.async_copy(dst_hbm.at[wid], dstv, sem0)

    zeros16 = jnp.zeros((16,), jnp.float32)

    def zbody(i, c):
        for k in range(D // 16):
            rows1[i, pl.ds(k * 16, 16)] = zeros16
        return c

    lax.fori_loop(0, CH, zbody, 0)
    base = sid * SEG
    for r in range(SEG // CH):
        pltpu.sync_copy(rows1, acc_sh.at[pl.ds(base + r * CH, CH)])
    pltpu.make_async_copy(src_hbm.at[wid], srcv, sem0).wait()
    pltpu.make_async_copy(dst_hbm.at[wid], dstv, sem0).wait()
    plsc.subcore_barrier()

    def gidx(j):
        return srcv.at[pl.ds(j * CH, CH)]

    pltpu.async_copy(z_hbm.at[gidx(0)], rows0, sem0)

    def body(i, c):
        j0 = 2 * i
        pltpu.async_copy(z_hbm.at[gidx(j0 + 1)], rows1, sem0)
        pltpu.make_async_copy(z_hbm.at[gidx(j0)], rows0, sem0).wait()
        pltpu.sync_copy(rows0, acc_sh.at[dstv.at[j0]], add=True)
        pltpu.async_copy(z_hbm.at[gidx(j0 + 2)], rows0, sem0)
        pltpu.make_async_copy(z_hbm.at[gidx(j0 + 1)], rows1, sem0).wait()
        pltpu.sync_copy(rows1, acc_sh.at[dstv.at[j0 + 1]], add=True)
        return c

    lax.fori_loop(0, (NCH - 1) // 2, body, 0)
    pltpu.make_async_copy(z_hbm.at[gidx(NCH - 1)], rows0, sem0).wait()
    pltpu.sync_copy(rows0, acc_sh.at[dstv.at[NCH - 1]], add=True)
    plsc.subcore_barrier()
    pltpu.sync_copy(acc_sh.at[pl.ds(base, SEG)],
                    out_hbm.at[cid, pl.ds(base, SEG)])


def _dinv_of(d0_ref, d1_ref):
    return lax.rsqrt(d0_ref[...] + d1_ref[...] + 1.0)


def _zmat_body(x_ref, w_ref, d0_ref, d1_ref, z_ref):
    xw = jnp.dot(x_ref[...], w_ref[...], preferred_element_type=jnp.float32)
    z_ref[...] = xw * _dinv_of(d0_ref, d1_ref)


def _zmat(x, W, d0_col, d1_col):
    return pl.pallas_call(
        _zmat_body,
        grid=(RB,),
        in_specs=[
            pl.BlockSpec((BR, D), lambda i: (i, 0)),
            pl.BlockSpec((D, D), lambda i: (0, 0)),
            pl.BlockSpec((BR, 1), lambda i: (i, 0)),
            pl.BlockSpec((BR, 1), lambda i: (i, 0)),
        ],
        out_specs=pl.BlockSpec((BR, D), lambda i: (i, 0)),
        out_shape=jax.ShapeDtypeStruct((N, D), jnp.float32),
    )(x, W, d0_col, d1_col)


def _comb_body(acc_ref, z_ref, d0_ref, d1_ref, b_ref, w_ref, z2_ref):
    dinv = _dinv_of(d0_ref, d1_ref)
    h = jnp.maximum(
        (acc_ref[0] + acc_ref[1] + z_ref[...]) * dinv + b_ref[...], 0.0)
    z2_ref[...] = jnp.dot(h, w_ref[...],
                          preferred_element_type=jnp.float32) * dinv


def _comb(accp, z, d0_col, d1_col, b_row, W):
    return pl.pallas_call(
        _comb_body,
        grid=(RB,),
        in_specs=[
            pl.BlockSpec((NC, BR, D), lambda i: (0, i, 0)),
            pl.BlockSpec((BR, D), lambda i: (i, 0)),
            pl.BlockSpec((BR, 1), lambda i: (i, 0)),
            pl.BlockSpec((BR, 1), lambda i: (i, 0)),
            pl.BlockSpec((1, D), lambda i: (0, 0)),
            pl.BlockSpec((D, D), lambda i: (0, 0)),
        ],
        out_specs=pl.BlockSpec((BR, D), lambda i: (i, 0)),
        out_shape=jax.ShapeDtypeStruct((N, D), jnp.float32),
    )(accp, z, d0_col, d1_col, b_row, W)


def _fin_body(acc_ref, z_ref, d0_ref, d1_ref, b_ref, batch_ref, wfc_ref,
              bfc_ref, logits_ref, prob_ref, pooled_s, counts_s):
    i = pl.program_id(0)

    @pl.when(i == 0)
    def _():
        pooled_s[...] = jnp.zeros_like(pooled_s)
        counts_s[...] = jnp.zeros_like(counts_s)

    dinv = _dinv_of(d0_ref, d1_ref)
    h = jnp.maximum(
        (acc_ref[0] + acc_ref[1] + z_ref[...]) * dinv + b_ref[...], 0.0)
    onehot = (batch_ref[...] == lax.broadcasted_iota(jnp.int32, (1, G), 1)
              ).astype(jnp.float32)
    pooled_s[...] += lax.dot_general(onehot, h, (((0,), (0,)), ((), ())),
                                     preferred_element_type=jnp.float32)
    counts_s[...] += lax.dot_general(onehot, jnp.ones((BR, 1), jnp.float32),
                                     (((0,), (0,)), ((), ())),
                                     preferred_element_type=jnp.float32)

    @pl.when(i == pl.num_programs(0) - 1)
    def _():
        pooled = pooled_s[...] / jnp.maximum(counts_s[...], 1.0)
        logits = jnp.dot(pooled, wfc_ref[...],
                         preferred_element_type=jnp.float32) + bfc_ref[...]
        logits_ref[...] = logits
        m = jnp.max(logits, axis=1, keepdims=True)
        e = jnp.exp(logits - m)
        prob_ref[...] = e / jnp.sum(e, axis=1, keepdims=True)


def _final(accp, z, d0_col, d1_col, b_row, batch_col, Wfc, bfc_row):
    return pl.pallas_call(
        _fin_body,
        grid=(RB,),
        in_specs=[
            pl.BlockSpec((NC, BR, D), lambda i: (0, i, 0)),
            pl.BlockSpec((BR, D), lambda i: (i, 0)),
            pl.BlockSpec((BR, 1), lambda i: (i, 0)),
            pl.BlockSpec((BR, 1), lambda i: (i, 0)),
            pl.BlockSpec((1, D), lambda i: (0, 0)),
            pl.BlockSpec((BR, 1), lambda i: (i, 0)),
            pl.BlockSpec((D, 2), lambda i: (0, 0)),
            pl.BlockSpec((1, 2), lambda i: (0, 0)),
        ],
        out_specs=[
            pl.BlockSpec((G, 2), lambda i: (0, 0)),
            pl.BlockSpec((G, 2), lambda i: (0, 0)),
        ],
        out_shape=[
            jax.ShapeDtypeStruct((G, 2), jnp.float32),
            jax.ShapeDtypeStruct((G, 2), jnp.float32),
        ],
        scratch_shapes=[
            pltpu.VMEM((G, D), jnp.float32),
            pltpu.VMEM((G, 1), jnp.float32),
        ],
    )(accp, z, d0_col, d1_col, b_row, batch_col, Wfc, bfc_row)


def kernel(x, edge_index, batch, W1, b1, W2, b2, Wfc, bfc):
    src2 = edge_index[0].reshape(NW, EPW)
    dst3 = edge_index[1].reshape(NW, NCH, CH)

    degp = _deg(dst3)
    d0_col = degp[:N].reshape(N, 1)
    d1_col = degp[NP:NP + N].reshape(N, 1)

    z1 = _zmat(x, W1, d0_col, d1_col)
    accp1 = _agg(z1, src2, dst3)
    z2 = _comb(accp1, z1, d0_col, d1_col, b1.reshape(1, D), W2)
    accp2 = _agg(z2, src2, dst3)
    logits, y_prob = _final(accp2, z2, d0_col, d1_col,
                            b2.reshape(1, D), batch.reshape(N, 1),
                            Wfc, bfc.reshape(1, 2))
    return (logits, y_prob)


# speedup vs baseline: 31.2355x; 1.0009x over previous
import functools
import jax
import jax.numpy as jnp
from jax import lax
from jax.experimental import pallas as pl
from jax.experimental.pallas import tpu as pltpu
from jax.experimental.pallas import tpu_sc as plsc

N = 10000
E = 320000
D = 128
G = 64
NC = 2
NS = 16
NW = NC * NS
EPW = E // NW
CH = 80
NCH = EPW // CH
NP = 10240
SEG = NP // NS
RB = 10
BR = N // RB

_mesh = plsc.VectorSubcoreMesh(core_axis_name="c", subcore_axis_name="s")


@functools.partial(
    pl.kernel,
    out_type=jax.ShapeDtypeStruct((NC * NP,), jnp.float32),
    mesh=_mesh,
    scratch_types=[
        pltpu.VMEM((NCH, CH), jnp.int32),
        pltpu.VMEM((CH,), jnp.float32),
        pltpu.VMEM((SEG,), jnp.float32),
        pltpu.VMEM_SHARED((NP,), jnp.float32),
        pltpu.SemaphoreType.DMA,
    ],
)
def _deg(dst_hbm, out_hbm, idx_v, ones_v, zero_v, acc_sh, sem):
    cid = lax.axis_index("c")
    sid = lax.axis_index("s")
    wid = sid * NC + cid
    ones16 = jnp.ones((16,), jnp.float32)
    zeros16 = jnp.zeros((16,), jnp.float32)
    for i in range(CH // 16):
        ones_v[pl.ds(i * 16, 16)] = ones16

    def zbody(i, c):
        zero_v[pl.ds(i * 16, 16)] = zeros16
        return c

    pltpu.async_copy(dst_hbm.at[wid], idx_v, sem)
    lax.fori_loop(0, SEG // 16, zbody, 0)
    pltpu.sync_copy(zero_v, acc_sh.at[pl.ds(sid * SEG, SEG)])
    pltpu.make_async_copy(dst_hbm.at[wid], idx_v, sem).wait()
    plsc.subcore_barrier()

    def body(j, c):
        pltpu.sync_copy(ones_v, acc_sh.at[idx_v.at[j]], add=True)
        return c

    lax.fori_loop(0, NCH, body, 0)
    plsc.subcore_barrier()
    pltpu.sync_copy(acc_sh.at[pl.ds(sid * SEG, SEG)],
                    out_hbm.at[pl.ds(cid * NP + sid * SEG, SEG)])


@functools.partial(
    pl.kernel,
    out_type=jax.ShapeDtypeStruct((NC, NP, D), jnp.float32),
    mesh=_mesh,
    scratch_types=[
        pltpu.VMEM((EPW,), jnp.int32),
        pltpu.VMEM((NCH, CH), jnp.int32),
        pltpu.VMEM((CH, D), jnp.float32),
        pltpu.VMEM((CH, D), jnp.float32),
        pltpu.VMEM_SHARED((NP, D), jnp.float32),
        pltpu.SemaphoreType.DMA,
    ],
)
def _agg(z_hbm, src_hbm, dst_hbm, out_hbm, srcv, dstv, rows0, rows1, acc_sh,
         sem0):
    cid = lax.axis_index("c")
    sid = lax.axis_index("s")
    wid = sid * NC + cid
    pltpu.async_copy(src_hbm.at[wid], srcv, sem0)
    pltpu.async_copy(dst_hbm.at[wid], dstv, sem0)

    zeros16 = jnp.zeros((16,), jnp.float32)

    def zbody(i, c):
        for k in range(D // 16):
            rows1[i, pl.ds(k * 16, 16)] = zeros16
        return c

    lax.fori_loop(0, CH, zbody, 0)
    base = sid * SEG
    for r in range(SEG // CH):
        pltpu.sync_copy(rows1, acc_sh.at[pl.ds(base + r * CH, CH)])
    pltpu.make_async_copy(src_hbm.at[wid], srcv, sem0).wait()
    pltpu.make_async_copy(dst_hbm.at[wid], dstv, sem0).wait()
    plsc.subcore_barrier()

    def gidx(j):
        return srcv.at[pl.ds(j * CH, CH)]

    pltpu.async_copy(z_hbm.at[gidx(0)], rows0, sem0)

    def body(i, c):
        j0 = 2 * i
        pltpu.async_copy(z_hbm.at[gidx(j0 + 1)], rows1, sem0)
        pltpu.make_async_copy(z_hbm.at[gidx(j0)], rows0, sem0).wait()
        pltpu.sync_copy(rows0, acc_sh.at[dstv.at[j0]], add=True)
        pltpu.async_copy(z_hbm.at[gidx(j0 + 2)], rows0, sem0)
        pltpu.make_async_copy(z_hbm.at[gidx(j0 + 1)], rows1, sem0).wait()
        pltpu.sync_copy(rows1, acc_sh.at[dstv.at[j0 + 1]], add=True)
        return c

    lax.fori_loop(0, (NCH - 1) // 2, body, 0)
    pltpu.make_async_copy(z_hbm.at[gidx(NCH - 1)], rows0, sem0).wait()
    pltpu.sync_copy(rows0, acc_sh.at[dstv.at[NCH - 1]], add=True)
    plsc.subcore_barrier()
    pltpu.sync_copy(acc_sh.at[pl.ds(base, SEG)],
                    out_hbm.at[cid, pl.ds(base, SEG)])


def _dinv_of(d0_ref, d1_ref):
    return lax.rsqrt(d0_ref[...] + d1_ref[...] + 1.0)


def _zmat_body(x_ref, w_ref, d0_ref, d1_ref, z_ref):
    xw = jnp.dot(x_ref[...], w_ref[...], preferred_element_type=jnp.float32)
    z_ref[...] = xw * _dinv_of(d0_ref, d1_ref)


def _zmat(x, W, d0_col, d1_col):
    return pl.pallas_call(
        _zmat_body,
        grid=(RB,),
        in_specs=[
            pl.BlockSpec((BR, D), lambda i: (i, 0)),
            pl.BlockSpec((D, D), lambda i: (0, 0)),
            pl.BlockSpec((BR, 1), lambda i: (i, 0)),
            pl.BlockSpec((BR, 1), lambda i: (i, 0)),
        ],
        out_specs=pl.BlockSpec((BR, D), lambda i: (i, 0)),
        out_shape=jax.ShapeDtypeStruct((N, D), jnp.float32),
    )(x, W, d0_col, d1_col)


def _comb_body(acc_ref, z_ref, d0_ref, d1_ref, b_ref, w_ref, z2_ref):
    dinv = _dinv_of(d0_ref, d1_ref)
    h = jnp.maximum(
        (acc_ref[0] + acc_ref[1] + z_ref[...]) * dinv + b_ref[...], 0.0)
    z2_ref[...] = jnp.dot(h, w_ref[...],
                          preferred_element_type=jnp.float32) * dinv


def _comb(accp, z, d0_col, d1_col, b_row, W):
    return pl.pallas_call(
        _comb_body,
        grid=(RB,),
        in_specs=[
            pl.BlockSpec((NC, BR, D), lambda i: (0, i, 0)),
            pl.BlockSpec((BR, D), lambda i: (i, 0)),
            pl.BlockSpec((BR, 1), lambda i: (i, 0)),
            pl.BlockSpec((BR, 1), lambda i: (i, 0)),
            pl.BlockSpec((1, D), lambda i: (0, 0)),
            pl.BlockSpec((D, D), lambda i: (0, 0)),
        ],
        out_specs=pl.BlockSpec((BR, D), lambda i: (i, 0)),
        out_shape=jax.ShapeDtypeStruct((N, D), jnp.float32),
    )(accp, z, d0_col, d1_col, b_row, W)


def _fin_body(acc_ref, z_ref, d0_ref, d1_ref, b_ref, batch_ref, wfc_ref,
              bfc_ref, logits_ref, prob_ref, pooled_s, counts_s):
    i = pl.program_id(0)

    @pl.when(i == 0)
    def _():
        pooled_s[...] = jnp.zeros_like(pooled_s)
        counts_s[...] = jnp.zeros_like(counts_s)

    dinv = _dinv_of(d0_ref, d1_ref)
    h = jnp.maximum(
        (acc_ref[0] + acc_ref[1] + z_ref[...]) * dinv + b_ref[...], 0.0)
    onehot = (batch_ref[...] == lax.broadcasted_iota(jnp.int32, (1, G), 1)
              ).astype(jnp.float32)
    pooled_s[...] += lax.dot_general(onehot, h, (((0,), (0,)), ((), ())),
                                     preferred_element_type=jnp.float32)
    counts_s[...] += lax.dot_general(onehot, jnp.ones((BR, 1), jnp.float32),
                                     (((0,), (0,)), ((), ())),
                                     preferred_element_type=jnp.float32)

    @pl.when(i == pl.num_programs(0) - 1)
    def _():
        pooled = pooled_s[...] / jnp.maximum(counts_s[...], 1.0)
        logits = jnp.dot(pooled, wfc_ref[...],
                         preferred_element_type=jnp.float32) + bfc_ref[...]
        logits_ref[...] = logits
        m = jnp.max(logits, axis=1, keepdims=True)
        e = jnp.exp(logits - m)
        prob_ref[...] = e / jnp.sum(e, axis=1, keepdims=True)


def _final(accp, z, d0_col, d1_col, b_row, batch_col, Wfc, bfc_row):
    return pl.pallas_call(
        _fin_body,
        grid=(RB,),
        in_specs=[
            pl.BlockSpec((NC, BR, D), lambda i: (0, i, 0)),
            pl.BlockSpec((BR, D), lambda i: (i, 0)),
            pl.BlockSpec((BR, 1), lambda i: (i, 0)),
            pl.BlockSpec((BR, 1), lambda i: (i, 0)),
            pl.BlockSpec((1, D), lambda i: (0, 0)),
            pl.BlockSpec((BR, 1), lambda i: (i, 0)),
            pl.BlockSpec((D, 2), lambda i: (0, 0)),
            pl.BlockSpec((1, 2), lambda i: (0, 0)),
        ],
        out_specs=[
            pl.BlockSpec((G, 2), lambda i: (0, 0)),
            pl.BlockSpec((G, 2), lambda i: (0, 0)),
        ],
        out_shape=[
            jax.ShapeDtypeStruct((G, 2), jnp.float32),
            jax.ShapeDtypeStruct((G, 2), jnp.float32),
        ],
        scratch_shapes=[
            pltpu.VMEM((G, D), jnp.float32),
            pltpu.VMEM((G, 1), jnp.float32),
        ],
    )(accp, z, d0_col, d1_col, b_row, batch_col, Wfc, bfc_row)


def kernel(x, edge_index, batch, W1, b1, W2, b2, Wfc, bfc):
    src2 = edge_index[0].reshape(NW, EPW)
    dst3 = edge_index[1].reshape(NW, NCH, CH)

    degp = _deg(dst3)
    d0_col = degp[:N].reshape(N, 1)
    d1_col = degp[NP:NP + N].reshape(N, 1)

    z1 = _zmat(x, W1, d0_col, d1_col)
    accp1 = _agg(z1, src2, dst3)
    z2 = _comb(accp1, z1, d0_col, d1_col, b1.reshape(1, D), W2)
    accp2 = _agg(z2, src2, dst3)
    logits, y_prob = _final(accp2, z2, d0_col, d1_col,
                            b2.reshape(1, D), batch.reshape(N, 1),
                            Wfc, bfc.reshape(1, 2))
    return (logits, y_prob)
